# baseline (speedup 1.0000x reference)
import jax
import jax.numpy as jnp
from jax.experimental import pallas as pl
from jax.experimental.pallas import tpu as pltpu

_LANES = 128
_SLOT = 136
_GAP = 4
_NEG = -1.0e30
_CHAINS = 4


def _lshift(h, s):
    if s == 0:
        return h
    if s > 0:
        return jnp.pad(h[:, s:], ((0, 0), (0, s)))
    return jnp.pad(h[:, :s], ((0, 0), (-s, 0)))


def _chain(xh, ind, w0, w1, w2, pmh):
    cols0 = jnp.concatenate([_lshift(xh, k - 3) for k in range(7)] + [ind],
                            axis=0)
    a0 = jnp.dot(w0, cols0, preferred_element_type=jnp.float32)
    h0 = jnp.maximum(a0, 0.0).astype(jnp.bfloat16)

    cols1 = jnp.concatenate([_lshift(h0, k - 2) for k in range(5)] + [ind],
                            axis=0)
    a1 = jnp.dot(w1, cols1, preferred_element_type=jnp.float32)
    h1 = jnp.maximum(a1, 0.0).astype(jnp.bfloat16)

    cols2 = jnp.concatenate([_lshift(h1, k - 1) for k in range(3)]
                            + [ind[:2]], axis=0)
    a2 = jnp.dot(w2, cols2, preferred_element_type=jnp.float32)
    h2 = jnp.maximum(a2, 0.0).astype(jnp.bfloat16)
    return jnp.dot(h2, pmh, preferred_element_type=jnp.float32)


def _fcn_kernel(x_ref, w0_ref, w1_ref, w2_ref, pool_ref, fcw_ref, fcb_ref,
                o_ref):
    NL = x_ref.shape[1]
    half = NL // _CHAINS
    pos = jax.lax.broadcasted_iota(jnp.int32, (3, half), 1) % _SLOT
    live = (pos >= _GAP) & (pos < _SLOT - _GAP)
    sel = live ^ (jax.lax.broadcasted_iota(jnp.int32, (3, half), 0) == 2)
    ind = jnp.where(sel, 1.0, 0.0).astype(jnp.bfloat16)

    xb = jnp.pad(x_ref[...], ((0, 16 - x_ref.shape[0]), (0, 0)))
    pooled = jnp.concatenate(
        [_chain(xb[:, i * half:(i + 1) * half], ind, w0_ref[...], w1_ref[...],
                w2_ref[...], pool_ref[...]) for i in range(_CHAINS)], axis=1)
    out = jax.lax.dot_general(pooled, fcw_ref[...], (((0,), (0,)), ((), ())),
                              preferred_element_type=jnp.float32)
    o_ref[0] = out + fcb_ref[...]


def kernel(conv0_w, conv0_scale, conv0_shift, conv1_w, conv1_scale,
           conv1_shift, conv2_w, conv2_scale, conv2_shift, fc_w, fc_b, x):
    N, cin, L = x.shape
    b_tile = 256
    num_tiles = pl.cdiv(N, b_tile)
    n_pad = num_tiles * b_tile
    NL = b_tile * _SLOT

    xt = jnp.transpose(x, (1, 0, 2))
    xt = jnp.pad(xt, ((0, 0), (0, n_pad - N), (_GAP, _GAP)))
    xhi = xt.astype(jnp.bfloat16)
    xlo = (xt - xhi.astype(jnp.float32)).astype(jnp.bfloat16)
    xg = jnp.concatenate([xhi, xlo], axis=0).reshape(2 * cin, n_pad * _SLOT)

    def bias_gap_cols(w, t, neg_gap):
        thi = t.astype(jnp.bfloat16).astype(jnp.float32)
        cols = [w, thi.reshape(-1, 1), (t - thi).reshape(-1, 1)]
        if neg_gap:
            cols.append(jnp.full((t.shape[0], 1), _NEG, jnp.float32))
        return jnp.concatenate(cols, axis=1)

    c0out = conv0_w.shape[2]
    w0 = conv0_w * conv0_scale[None, None, :]
    w0 = jnp.concatenate([w0, w0], axis=1)
    w0 = jnp.pad(w0, ((0, 0), (0, 16 - 2 * cin), (0, 0)))
    w0 = jnp.transpose(w0, (2, 0, 1)).reshape(c0out, 7 * 16)
    w0 = bias_gap_cols(w0, conv0_shift, True).astype(jnp.bfloat16)

    w1 = conv1_w * conv1_scale[None, None, :]
    w1 = jnp.transpose(w1, (2, 0, 1)).reshape(_LANES, 5 * c0out)
    w1 = bias_gap_cols(w1, conv1_shift, True).astype(jnp.bfloat16)

    w2 = conv2_w * conv2_scale[None, None, :]
    w2 = jnp.transpose(w2, (2, 0, 1)).reshape(64, 3 * _LANES)
    w2 = bias_gap_cols(w2, conv2_shift, False).astype(jnp.bfloat16)

    ar = jnp.arange(NL // _CHAINS)
    posv = ar % _SLOT
    sig = (posv >= _GAP) & (posv < _SLOT - _GAP)
    blk = ar // _SLOT
    pm = (sig[:, None] & (blk[:, None] == jnp.arange(b_tile // _CHAINS)[None, :]))
    pm = (pm.astype(jnp.float32) / L).astype(jnp.bfloat16)

    fcw = fc_w.astype(jnp.float32)
    fcb = fc_b.reshape(1, 2)

    consts = [w0, w1, w2, pm, fcw, fcb]
    out = pl.pallas_call(
        _fcn_kernel,
        out_shape=jax.ShapeDtypeStruct((num_tiles, b_tile, 2), jnp.float32),
        grid=(num_tiles,),
        in_specs=[pl.BlockSpec((2 * cin, NL), lambda n: (0, n))]
        + [pl.BlockSpec(a.shape, lambda n, nd=a.ndim: (0,) * nd) for a in consts],
        out_specs=pl.BlockSpec((1, b_tile, 2), lambda n: (n, 0, 0)),
        compiler_params=pltpu.CompilerParams(
            dimension_semantics=("parallel",)),
    )(xg, *consts)
    return out.reshape(n_pad, 2)[:N]


# speedup vs baseline: 1.3760x; 1.3760x over previous
import jax
import jax.numpy as jnp
from jax.experimental import pallas as pl
from jax.experimental.pallas import tpu as pltpu

_LANES = 128
_SLOT = 136
_GAP = 4
_NEG = -1.0e30
_CHAINS = 4


def _lshift(h, s):
    if s == 0:
        return h
    if s > 0:
        return jnp.pad(h[:, s:], ((0, 0), (0, s)))
    return jnp.pad(h[:, :s], ((0, 0), (-s, 0)))


def _chain(xh, ind, w0, w1, w2, pmh):
    cols0 = jnp.concatenate([_lshift(xh, k - 3) for k in range(7)] + [ind],
                            axis=0)
    a0 = jnp.dot(w0, cols0, preferred_element_type=jnp.float32)
    h0 = jnp.maximum(a0, 0.0).astype(jnp.bfloat16)

    cols1 = jnp.concatenate([_lshift(h0, k - 2) for k in range(5)] + [ind],
                            axis=0)
    a1 = jnp.dot(w1, cols1, preferred_element_type=jnp.float32)
    h1 = jnp.maximum(a1, 0.0).astype(jnp.bfloat16)

    cols2 = jnp.concatenate([_lshift(h1, k - 1) for k in range(3)]
                            + [ind[:1]], axis=0)
    a2 = jnp.dot(w2, cols2, preferred_element_type=jnp.float32)
    h2 = jnp.maximum(a2, 0.0).astype(jnp.bfloat16)
    return jnp.dot(h2, pmh, preferred_element_type=jnp.float32)


def _fcn_kernel(x_ref, w0_ref, w1_ref, w2_ref, pool_ref, fcw_ref, fcb_ref,
                o_ref):
    NL = x_ref.shape[1]
    half = NL // _CHAINS
    pos = jax.lax.broadcasted_iota(jnp.int32, (2, half), 1) % _SLOT
    live = (pos >= _GAP) & (pos < _SLOT - _GAP)
    sel = live ^ (jax.lax.broadcasted_iota(jnp.int32, (2, half), 0) == 1)
    ind = jnp.where(sel, 1.0, 0.0).astype(jnp.bfloat16)

    xb = jnp.pad(x_ref[...], ((0, 16 - x_ref.shape[0]), (0, 0)))
    pooled = jnp.concatenate(
        [_chain(xb[:, i * half:(i + 1) * half], ind, w0_ref[...], w1_ref[...],
                w2_ref[...], pool_ref[...]) for i in range(_CHAINS)], axis=1)
    out = jax.lax.dot_general(pooled, fcw_ref[...], (((0,), (0,)), ((), ())),
                              preferred_element_type=jnp.float32)
    o_ref[0] = out + fcb_ref[...]


def kernel(conv0_w, conv0_scale, conv0_shift, conv1_w, conv1_scale,
           conv1_shift, conv2_w, conv2_scale, conv2_shift, fc_w, fc_b, x):
    N, cin, L = x.shape
    b_tile = 256
    num_tiles = pl.cdiv(N, b_tile)
    n_pad = num_tiles * b_tile
    NL = b_tile * _SLOT

    xt = jnp.transpose(x, (1, 0, 2))
    xt = jnp.pad(xt, ((0, 0), (0, n_pad - N), (_GAP, _GAP)))
    xg = xt.reshape(cin, n_pad * _SLOT).astype(jnp.bfloat16)

    def bias_gap_cols(w, t, neg_gap):
        c1 = t.reshape(-1, 1)
        cols = [w, c1]
        if neg_gap:
            cols.append(jnp.full_like(c1, _NEG))
        return jnp.concatenate(cols, axis=1)

    c0out = conv0_w.shape[2]
    w0 = conv0_w * conv0_scale[None, None, :]
    w0 = jnp.pad(w0, ((0, 0), (0, 16 - cin), (0, 0)))
    w0 = jnp.transpose(w0, (2, 0, 1)).reshape(c0out, 7 * 16)
    w0 = bias_gap_cols(w0, conv0_shift, True).astype(jnp.bfloat16)

    w1 = conv1_w * conv1_scale[None, None, :]
    w1 = jnp.transpose(w1, (2, 0, 1)).reshape(_LANES, 5 * c0out)
    w1 = bias_gap_cols(w1, conv1_shift, True).astype(jnp.bfloat16)

    w2 = conv2_w * conv2_scale[None, None, :]
    w2 = jnp.transpose(w2, (2, 0, 1)).reshape(64, 3 * _LANES)
    w2 = bias_gap_cols(w2, conv2_shift, False).astype(jnp.bfloat16)

    ar = jnp.arange(NL // _CHAINS)
    posv = ar % _SLOT
    sig = (posv >= _GAP) & (posv < _SLOT - _GAP)
    blk = ar // _SLOT
    pm = (sig[:, None] & (blk[:, None] == jnp.arange(b_tile // _CHAINS)[None, :]))
    pm = (pm.astype(jnp.float32) / L).astype(jnp.bfloat16)

    fcw = fc_w.astype(jnp.float32)
    fcb = fc_b.reshape(1, 2)

    consts = [w0, w1, w2, pm, fcw, fcb]
    out = pl.pallas_call(
        _fcn_kernel,
        out_shape=jax.ShapeDtypeStruct((num_tiles, b_tile, 2), jnp.float32),
        grid=(num_tiles,),
        in_specs=[pl.BlockSpec((cin, NL), lambda n: (0, n))]
        + [pl.BlockSpec(a.shape, lambda n, nd=a.ndim: (0,) * nd) for a in consts],
        out_specs=pl.BlockSpec((1, b_tile, 2), lambda n: (n, 0, 0)),
        compiler_params=pltpu.CompilerParams(
            dimension_semantics=("parallel",)),
    )(xg, *consts)
    return out.reshape(n_pad, 2)[:N]
